# Initial kernel scaffold; baseline (speedup 1.0000x reference)
import jax
import jax.numpy as jnp
from jax.experimental import pallas as pl


def kernel(x, gate_W, fc1_W, fc1_b, fc2_W, fc2_b):
    raise NotImplementedError("write your pallas kernel here")


# speedup vs baseline: 6.9923x; 6.9923x over previous
import functools

import jax
import jax.numpy as jnp
from jax import lax
from jax.experimental import pallas as pl
from jax.experimental.pallas import tpu as pltpu
from jax.experimental.pallas import tpu_sc as plsc

D_MODEL = 1024
D_FF = 4096
NE = 16
NTOK = 4096
TBLK = 512
NTBLK = NTOK // TBLK
G = 512
NBLK = 32
PAD_ROWS = NBLK * G
FF_T = 2048
NFT = D_FF // FF_T

F32 = jnp.float32
I32 = jnp.int32
DEFP = lax.Precision.DEFAULT


def _gate_kernel(x_ref, gw_ref, w0_ref, w1_ref, s0_ref, s1_ref, be_ref,
                 lb_ref, cnt, fsum, psum, offs):
    ph = pl.program_id(0)
    blk = pl.program_id(1)

    @pl.when(jnp.logical_and(ph == 0, blk == 0))
    def _init():
        cnt[...] = jnp.zeros_like(cnt)
        fsum[...] = jnp.zeros_like(fsum)
        psum[...] = jnp.zeros_like(psum)
        offs[...] = jnp.zeros_like(offs)

    x = x_ref[...]
    gw = gw_ref[...]
    logits = lax.dot_general(x, gw, (((1,), (1,)), ((), ())),
                             precision=DEFP, preferred_element_type=F32)

    lane = lax.broadcasted_iota(I32, (TBLK, NE), 1)
    l0 = jnp.max(logits, axis=1, keepdims=True)
    i0 = jnp.min(jnp.where(logits == l0, lane, NE), axis=1, keepdims=True)
    masked = jnp.where(lane == i0, -jnp.inf, logits)
    l1 = jnp.max(masked, axis=1, keepdims=True)
    i1 = jnp.min(jnp.where(masked == l1, lane, NE), axis=1, keepdims=True)

    e1 = jnp.exp(l1 - l0)
    w0_ref[...] = jnp.broadcast_to(1.0 / (1.0 + e1), (TBLK, 16)).astype(F32)
    w1_ref[...] = jnp.broadcast_to(e1 / (1.0 + e1), (TBLK, 16)).astype(F32)

    m0 = (lane == i0).astype(F32)
    m1 = (lane == i1).astype(F32)
    m01 = m0 + m1

    @pl.when(ph == 0)
    def _accum():
        probs = jnp.exp(logits - l0)
        probs = probs / jnp.sum(probs, axis=1, keepdims=True)
        psum[...] += jnp.sum(probs, axis=0, keepdims=True)
        fsum[...] += jnp.sum(m0, axis=0, keepdims=True)

    r = lax.broadcasted_iota(I32, (TBLK, TBLK), 0)
    c = lax.broadcasted_iota(I32, (TBLK, TBLK), 1)
    lt = (c < r).astype(F32)
    cum = lax.dot_general(lt, m01, (((1,), (0,)), ((), ())),
                          precision=DEFP, preferred_element_type=F32)

    base = cum + cnt[...] + offs[...]
    s0 = jnp.sum(base * m0, axis=1, keepdims=True)
    s1 = jnp.sum(base * m1, axis=1, keepdims=True)
    s0_ref[...] = s0.astype(I32)
    s1_ref[...] = s1.astype(I32)

    cnt[...] += jnp.sum(m01, axis=0, keepdims=True)

    @pl.when(jnp.logical_and(ph == 0, blk == NTBLK - 1))
    def _finish_phase0():
        lb_ref[...] = (NE / (NTOK * NTOK)) * jnp.sum(
            fsum[...] * psum[...], axis=1, keepdims=True)
        padded = jnp.ceil(cnt[...] * (1.0 / G)) * G
        i = lax.broadcasted_iota(I32, (NE, NE), 0)
        j = lax.broadcasted_iota(I32, (NE, NE), 1)
        lt16 = (i < j).astype(F32)
        offs[...] = lax.dot_general(padded, lt16, (((1,), (0,)), ((), ())),
                                    precision=DEFP, preferred_element_type=F32)
        rb = lax.broadcasted_iota(I32, (NBLK, NE), 0).astype(F32) * G
        cmp = (offs[...] <= rb).astype(F32)
        be_ref[...] = (jnp.sum(cmp, axis=1, keepdims=True) - 1.0).astype(I32)
        cnt[...] = jnp.zeros_like(cnt)


def _gating(xf, gate_W):
    return pl.pallas_call(
        _gate_kernel,
        grid=(2, NTBLK),
        in_specs=[
            pl.BlockSpec((TBLK, D_MODEL), lambda p, b: (b, 0)),
            pl.BlockSpec((NE, D_MODEL), lambda p, b: (0, 0)),
        ],
        out_specs=[
            pl.BlockSpec((TBLK, 16), lambda p, b: (b, 0)),
            pl.BlockSpec((TBLK, 16), lambda p, b: (b, 0)),
            pl.BlockSpec((TBLK, 1), lambda p, b: (b, 0)),
            pl.BlockSpec((TBLK, 1), lambda p, b: (b, 0)),
            pl.BlockSpec((NBLK, 1), lambda p, b: (0, 0)),
            pl.BlockSpec((1, 1), lambda p, b: (0, 0)),
        ],
        out_shape=[
            jax.ShapeDtypeStruct((NTOK, 16), F32),
            jax.ShapeDtypeStruct((NTOK, 16), F32),
            jax.ShapeDtypeStruct((NTOK, 1), I32),
            jax.ShapeDtypeStruct((NTOK, 1), I32),
            jax.ShapeDtypeStruct((NBLK, 1), I32),
            jax.ShapeDtypeStruct((1, 1), F32),
        ],
        scratch_shapes=[
            pltpu.VMEM((1, NE), F32),
            pltpu.VMEM((1, NE), F32),
            pltpu.VMEM((1, NE), F32),
            pltpu.VMEM((1, NE), F32),
        ],
    )(xf, gate_W)


def _make_dispatch_kernel():
    mesh = plsc.VectorSubcoreMesh(core_axis_name="c", subcore_axis_name="s")
    NW = 32
    TOK_W = NTOK // NW
    CH = 16

    @functools.partial(
        pl.kernel, mesh=mesh,
        out_type=jax.ShapeDtypeStruct((PAD_ROWS, D_MODEL), F32),
        scratch_types=[
            pltpu.VMEM((CH,), I32),
            pltpu.VMEM((CH,), I32),
            pltpu.VMEM((CH, D_MODEL), F32),
            pltpu.SemaphoreType.DMA,
            pltpu.SemaphoreType.DMA,
        ],
    )
    def k(xf_hbm, s0_hbm, s1_hbm, xs_hbm, i0v, i1v, rowsv, sem0, sem1):
        wid = lax.axis_index("s") * 2 + lax.axis_index("c")
        base = wid * TOK_W

        def body(ch, _):
            off = base + ch * CH
            pltpu.sync_copy(s0_hbm.at[pl.ds(off, CH)], i0v)
            pltpu.sync_copy(s1_hbm.at[pl.ds(off, CH)], i1v)
            pltpu.sync_copy(xf_hbm.at[pl.ds(off, CH)], rowsv)
            sl0 = i0v[...]
            sl1 = i1v[...]
            cp0 = pltpu.async_copy(rowsv, xs_hbm.at[sl0], sem0)
            cp1 = pltpu.async_copy(rowsv, xs_hbm.at[sl1], sem1)
            cp0.wait()
            cp1.wait()
            return 0
        lax.fori_loop(0, TOK_W // CH, body, 0)

    return k


_dispatch_kernel = functools.cache(_make_dispatch_kernel)


def _ffn_kernel(be_ref, xs_ref, w1_ref, b1_ref, w2_ref, b2_ref, eo_ref):
    f = pl.program_id(1)
    x = xs_ref[...]
    w1 = w1_ref[0]
    h = lax.dot_general(x, w1, (((1,), (1,)), ((), ())),
                        precision=DEFP, preferred_element_type=F32)
    h = h + b1_ref[0, 0]
    h = 0.5 * h * (1.0 + lax.erf(h * (2.0 ** -0.5)))
    w2 = w2_ref[0]
    acc = lax.dot_general(h, w2, (((1,), (1,)), ((), ())),
                          precision=DEFP, preferred_element_type=F32)

    @pl.when(f == 0)
    def _first():
        eo_ref[...] = acc + b2_ref[0]

    @pl.when(f != 0)
    def _rest():
        eo_ref[...] += acc


def _ffn(block_expert, xs, fc1_W, fc1_b, fc2_W, fc2_b):
    return pl.pallas_call(
        _ffn_kernel,
        grid_spec=pltpu.PrefetchScalarGridSpec(
            num_scalar_prefetch=1,
            grid=(NBLK, NFT),
            in_specs=[
                pl.BlockSpec((G, D_MODEL), lambda b, f, s: (b, 0)),
                pl.BlockSpec((1, FF_T, D_MODEL), lambda b, f, s: (s[b, 0], f, 0)),
                pl.BlockSpec((1, 1, 1, FF_T), lambda b, f, s: (s[b, 0], f, 0, 0)),
                pl.BlockSpec((1, D_MODEL, FF_T), lambda b, f, s: (s[b, 0], 0, f)),
                pl.BlockSpec((1, 1, D_MODEL), lambda b, f, s: (s[b, 0], 0, 0)),
            ],
            out_specs=pl.BlockSpec((G, D_MODEL), lambda b, f, s: (b, 0)),
        ),
        out_shape=jax.ShapeDtypeStruct((PAD_ROWS, D_MODEL), F32),
    )(block_expert, xs, fc1_W, fc1_b.reshape(NE, NFT, 1, FF_T), fc2_W,
      fc2_b.reshape(NE, 1, D_MODEL))


def _make_combine_kernel():
    mesh = plsc.VectorSubcoreMesh(core_axis_name="c", subcore_axis_name="s")
    NW = 32
    TOK_W = NTOK // NW
    CH = 16

    @functools.partial(
        pl.kernel, mesh=mesh,
        out_type=jax.ShapeDtypeStruct((NTOK, D_MODEL), F32),
        scratch_types=[
            pltpu.VMEM((CH,), I32),
            pltpu.VMEM((CH,), I32),
            pltpu.VMEM((CH, 16), F32),
            pltpu.VMEM((CH, 16), F32),
            pltpu.VMEM((CH, D_MODEL), F32),
            pltpu.VMEM((CH, D_MODEL), F32),
            pltpu.SemaphoreType.DMA,
            pltpu.SemaphoreType.DMA,
        ],
    )
    def k(eo_hbm, s0_hbm, s1_hbm, w0_hbm, w1_hbm, out_hbm,
          i0v, i1v, w0v, w1v, b0v, b1v, sem0, sem1):
        wid = lax.axis_index("s") * 2 + lax.axis_index("c")
        base = wid * TOK_W

        def body(ch, _):
            off = base + ch * CH
            pltpu.sync_copy(s0_hbm.at[pl.ds(off, CH)], i0v)
            pltpu.sync_copy(s1_hbm.at[pl.ds(off, CH)], i1v)
            pltpu.sync_copy(w0_hbm.at[pl.ds(off, CH)], w0v)
            pltpu.sync_copy(w1_hbm.at[pl.ds(off, CH)], w1v)
            cp0 = pltpu.async_copy(eo_hbm.at[i0v], b0v, sem0)
            cp1 = pltpu.async_copy(eo_hbm.at[i1v], b1v, sem1)
            cp0.wait()
            cp1.wait()

            def add_row(i, _):
                wr0 = w0v[i, pl.ds(0, 16)]
                wr1 = w1v[i, pl.ds(0, 16)]

                def add_vec(j, _):
                    b0v[i, pl.ds(j * 16, 16)] = (
                        b0v[i, pl.ds(j * 16, 16)] * wr0
                        + b1v[i, pl.ds(j * 16, 16)] * wr1)
                    return 0
                lax.fori_loop(0, D_MODEL // 16, add_vec, 0, unroll=4)
                return 0
            lax.fori_loop(0, CH, add_row, 0)

            pltpu.sync_copy(b0v, out_hbm.at[pl.ds(off, CH)])
            return 0
        lax.fori_loop(0, TOK_W // CH, body, 0)

    return k


_combine_kernel = functools.cache(_make_combine_kernel)


def kernel(x, gate_W, fc1_W, fc1_b, fc2_W, fc2_b):
    B_, S_, D = x.shape
    xf = x.reshape(-1, D)

    w0, w1, s0, s1, be, lb = _gating(xf, gate_W)
    s0f = s0.reshape(-1)
    s1f = s1.reshape(-1)

    xs = _dispatch_kernel()(xf, s0f, s1f)
    eo = _ffn(be, xs, fc1_W, fc1_b, fc2_W, fc2_b)
    out = _combine_kernel()(eo, s0f, s1f, w0, w1)

    return out.reshape(B_, S_, D), lb[0, 0]


# speedup vs baseline: 7.2000x; 1.0297x over previous
import functools

import jax
import jax.numpy as jnp
from jax import lax
from jax.experimental import pallas as pl
from jax.experimental.pallas import tpu as pltpu
from jax.experimental.pallas import tpu_sc as plsc

D_MODEL = 1024
D_FF = 4096
NE = 16
NTOK = 4096
TBLK = 512
NTBLK = NTOK // TBLK
G = 512
NBLK = 32
PAD_ROWS = NBLK * G
FF_T = 2048
NFT = D_FF // FF_T

F32 = jnp.float32
I32 = jnp.int32
DEFP = lax.Precision.DEFAULT


def _gate_kernel(x_ref, gw_ref, w0_ref, w1_ref, s0_ref, s1_ref, be_ref,
                 lb_ref, cnt, fsum, psum, offs):
    ph = pl.program_id(0)
    blk = pl.program_id(1)

    @pl.when(jnp.logical_and(ph == 0, blk == 0))
    def _init():
        cnt[...] = jnp.zeros_like(cnt)
        fsum[...] = jnp.zeros_like(fsum)
        psum[...] = jnp.zeros_like(psum)
        offs[...] = jnp.zeros_like(offs)

    x = x_ref[...]
    gw = gw_ref[...]
    logits = lax.dot_general(x, gw, (((1,), (1,)), ((), ())),
                             precision=DEFP, preferred_element_type=F32)

    lane = lax.broadcasted_iota(I32, (TBLK, NE), 1)
    l0 = jnp.max(logits, axis=1, keepdims=True)
    i0 = jnp.min(jnp.where(logits == l0, lane, NE), axis=1, keepdims=True)
    masked = jnp.where(lane == i0, -jnp.inf, logits)
    l1 = jnp.max(masked, axis=1, keepdims=True)
    i1 = jnp.min(jnp.where(masked == l1, lane, NE), axis=1, keepdims=True)

    e1 = jnp.exp(l1 - l0)
    w0_ref[...] = jnp.broadcast_to(1.0 / (1.0 + e1), (TBLK, 16)).astype(F32)
    w1_ref[...] = jnp.broadcast_to(e1 / (1.0 + e1), (TBLK, 16)).astype(F32)

    m0 = (lane == i0).astype(F32)
    m1 = (lane == i1).astype(F32)
    m01 = m0 + m1

    @pl.when(ph == 0)
    def _accum():
        probs = jnp.exp(logits - l0)
        probs = probs / jnp.sum(probs, axis=1, keepdims=True)
        psum[...] += jnp.sum(probs, axis=0, keepdims=True)
        fsum[...] += jnp.sum(m0, axis=0, keepdims=True)

    r = lax.broadcasted_iota(I32, (TBLK, TBLK), 0)
    c = lax.broadcasted_iota(I32, (TBLK, TBLK), 1)
    lt = (c < r).astype(F32)
    cum = lax.dot_general(lt, m01, (((1,), (0,)), ((), ())),
                          precision=DEFP, preferred_element_type=F32)

    base = cum + cnt[...] + offs[...]
    s0 = jnp.sum(base * m0, axis=1, keepdims=True)
    s1 = jnp.sum(base * m1, axis=1, keepdims=True)
    s0_ref[...] = s0.astype(I32)
    s1_ref[...] = s1.astype(I32)

    cnt[...] += jnp.sum(m01, axis=0, keepdims=True)

    @pl.when(jnp.logical_and(ph == 0, blk == NTBLK - 1))
    def _finish_phase0():
        lb_ref[...] = (NE / (NTOK * NTOK)) * jnp.sum(
            fsum[...] * psum[...], axis=1, keepdims=True)
        padded = jnp.ceil(cnt[...] * (1.0 / G)) * G
        i = lax.broadcasted_iota(I32, (NE, NE), 0)
        j = lax.broadcasted_iota(I32, (NE, NE), 1)
        lt16 = (i < j).astype(F32)
        offs[...] = lax.dot_general(padded, lt16, (((1,), (0,)), ((), ())),
                                    precision=DEFP, preferred_element_type=F32)
        tp = jnp.sum(padded, axis=1, keepdims=True)
        rb = lax.broadcasted_iota(I32, (NBLK, NE), 0).astype(F32) * G
        rb = jnp.minimum(rb, tp - G)
        cmp = (offs[...] <= rb).astype(F32)
        be = (jnp.sum(cmp, axis=1, keepdims=True) - 1.0).astype(I32)
        used = (tp * (1.0 / G)).astype(I32)
        be_ref[...] = jnp.concatenate([be, used], axis=0)
        cnt[...] = jnp.zeros_like(cnt)


def _gating(xf, gate_W):
    return pl.pallas_call(
        _gate_kernel,
        grid=(2, NTBLK),
        in_specs=[
            pl.BlockSpec((TBLK, D_MODEL), lambda p, b: (b, 0)),
            pl.BlockSpec((NE, D_MODEL), lambda p, b: (0, 0)),
        ],
        out_specs=[
            pl.BlockSpec((TBLK, 16), lambda p, b: (b, 0)),
            pl.BlockSpec((TBLK, 16), lambda p, b: (b, 0)),
            pl.BlockSpec((TBLK, 1), lambda p, b: (b, 0)),
            pl.BlockSpec((TBLK, 1), lambda p, b: (b, 0)),
            pl.BlockSpec((NBLK + 1, 1), lambda p, b: (0, 0)),
            pl.BlockSpec((1, 1), lambda p, b: (0, 0)),
        ],
        out_shape=[
            jax.ShapeDtypeStruct((NTOK, 16), F32),
            jax.ShapeDtypeStruct((NTOK, 16), F32),
            jax.ShapeDtypeStruct((NTOK, 1), I32),
            jax.ShapeDtypeStruct((NTOK, 1), I32),
            jax.ShapeDtypeStruct((NBLK + 1, 1), I32),
            jax.ShapeDtypeStruct((1, 1), F32),
        ],
        scratch_shapes=[
            pltpu.VMEM((1, NE), F32),
            pltpu.VMEM((1, NE), F32),
            pltpu.VMEM((1, NE), F32),
            pltpu.VMEM((1, NE), F32),
        ],
    )(xf, gate_W)


def _make_dispatch_kernel():
    mesh = plsc.VectorSubcoreMesh(core_axis_name="c", subcore_axis_name="s")
    NW = 32
    TOK_W = NTOK // NW
    CH = 16

    @functools.partial(
        pl.kernel, mesh=mesh,
        out_type=jax.ShapeDtypeStruct((PAD_ROWS, D_MODEL), F32),
        scratch_types=[
            pltpu.VMEM((CH,), I32),
            pltpu.VMEM((CH,), I32),
            pltpu.VMEM((CH, D_MODEL), F32),
            pltpu.SemaphoreType.DMA,
            pltpu.SemaphoreType.DMA,
        ],
    )
    def k(xf_hbm, s0_hbm, s1_hbm, xs_hbm, i0v, i1v, rowsv, sem0, sem1):
        wid = lax.axis_index("s") * 2 + lax.axis_index("c")
        base = wid * TOK_W

        def body(ch, _):
            off = base + ch * CH
            pltpu.sync_copy(s0_hbm.at[pl.ds(off, CH)], i0v)
            pltpu.sync_copy(s1_hbm.at[pl.ds(off, CH)], i1v)
            pltpu.sync_copy(xf_hbm.at[pl.ds(off, CH)], rowsv)
            sl0 = i0v[...]
            sl1 = i1v[...]
            cp0 = pltpu.async_copy(rowsv, xs_hbm.at[sl0], sem0)
            cp1 = pltpu.async_copy(rowsv, xs_hbm.at[sl1], sem1)
            cp0.wait()
            cp1.wait()
            return 0
        lax.fori_loop(0, TOK_W // CH, body, 0)

    return k


_dispatch_kernel = functools.cache(_make_dispatch_kernel)


def _ffn_kernel(be_ref, xs_ref, w1_ref, b1_ref, w2_ref, b2_ref, eo_ref):
    b = pl.program_id(0)
    f = pl.program_id(1)

    @pl.when(b < be_ref[NBLK, 0])
    def _compute():
        x = xs_ref[...]
        w1 = w1_ref[0]
        h = lax.dot_general(x, w1, (((1,), (1,)), ((), ())),
                            precision=DEFP, preferred_element_type=F32)
        h = h + b1_ref[0, 0]
        h = 0.5 * h * (1.0 + lax.erf(h * (2.0 ** -0.5)))
        w2 = w2_ref[0]
        acc = lax.dot_general(h, w2, (((1,), (1,)), ((), ())),
                              precision=DEFP, preferred_element_type=F32)

        @pl.when(f == 0)
        def _first():
            eo_ref[...] = acc + b2_ref[0]

        @pl.when(f != 0)
        def _rest():
            eo_ref[...] += acc


def _ffn(block_expert, xs, fc1_W, fc1_b, fc2_W, fc2_b):
    return pl.pallas_call(
        _ffn_kernel,
        grid_spec=pltpu.PrefetchScalarGridSpec(
            num_scalar_prefetch=1,
            grid=(NBLK, NFT),
            in_specs=[
                pl.BlockSpec((G, D_MODEL),
                             lambda b, f, s: (jnp.where(b < s[NBLK, 0], b, 0), 0)),
                pl.BlockSpec((1, FF_T, D_MODEL), lambda b, f, s: (s[b, 0], f, 0)),
                pl.BlockSpec((1, 1, 1, FF_T), lambda b, f, s: (s[b, 0], f, 0, 0)),
                pl.BlockSpec((1, D_MODEL, FF_T), lambda b, f, s: (s[b, 0], 0, f)),
                pl.BlockSpec((1, 1, D_MODEL), lambda b, f, s: (s[b, 0], 0, 0)),
            ],
            out_specs=pl.BlockSpec((G, D_MODEL), lambda b, f, s: (b, 0)),
        ),
        out_shape=jax.ShapeDtypeStruct((PAD_ROWS, D_MODEL), F32),
    )(block_expert, xs, fc1_W, fc1_b.reshape(NE, NFT, 1, FF_T), fc2_W,
      fc2_b.reshape(NE, 1, D_MODEL))


def _make_combine_kernel():
    mesh = plsc.VectorSubcoreMesh(core_axis_name="c", subcore_axis_name="s")
    NW = 32
    TOK_W = NTOK // NW
    CH = 16

    @functools.partial(
        pl.kernel, mesh=mesh,
        out_type=jax.ShapeDtypeStruct((NTOK, D_MODEL), F32),
        scratch_types=[
            pltpu.VMEM((CH,), I32),
            pltpu.VMEM((CH,), I32),
            pltpu.VMEM((CH, 16), F32),
            pltpu.VMEM((CH, 16), F32),
            pltpu.VMEM((CH, D_MODEL), F32),
            pltpu.VMEM((CH, D_MODEL), F32),
            pltpu.SemaphoreType.DMA,
            pltpu.SemaphoreType.DMA,
        ],
    )
    def k(eo_hbm, s0_hbm, s1_hbm, w0_hbm, w1_hbm, out_hbm,
          i0v, i1v, w0v, w1v, b0v, b1v, sem0, sem1):
        wid = lax.axis_index("s") * 2 + lax.axis_index("c")
        base = wid * TOK_W

        def body(ch, _):
            off = base + ch * CH
            pltpu.sync_copy(s0_hbm.at[pl.ds(off, CH)], i0v)
            pltpu.sync_copy(s1_hbm.at[pl.ds(off, CH)], i1v)
            pltpu.sync_copy(w0_hbm.at[pl.ds(off, CH)], w0v)
            pltpu.sync_copy(w1_hbm.at[pl.ds(off, CH)], w1v)
            cp0 = pltpu.async_copy(eo_hbm.at[i0v], b0v, sem0)
            cp1 = pltpu.async_copy(eo_hbm.at[i1v], b1v, sem1)
            cp0.wait()
            cp1.wait()

            def add_row(i, _):
                wr0 = w0v[i, pl.ds(0, 16)]
                wr1 = w1v[i, pl.ds(0, 16)]

                def add_vec(j, _):
                    b0v[i, pl.ds(j * 16, 16)] = (
                        b0v[i, pl.ds(j * 16, 16)] * wr0
                        + b1v[i, pl.ds(j * 16, 16)] * wr1)
                    return 0
                lax.fori_loop(0, D_MODEL // 16, add_vec, 0, unroll=4)
                return 0
            lax.fori_loop(0, CH, add_row, 0)

            pltpu.sync_copy(b0v, out_hbm.at[pl.ds(off, CH)])
            return 0
        lax.fori_loop(0, TOK_W // CH, body, 0)

    return k


_combine_kernel = functools.cache(_make_combine_kernel)


def kernel(x, gate_W, fc1_W, fc1_b, fc2_W, fc2_b):
    B_, S_, D = x.shape
    xf = x.reshape(-1, D)

    w0, w1, s0, s1, be, lb = _gating(xf, gate_W)
    s0f = s0.reshape(-1)
    s1f = s1.reshape(-1)

    xs = _dispatch_kernel()(xf, s0f, s1f)
    eo = _ffn(be, xs, fc1_W, fc1_b, fc2_W, fc2_b)
    out = _combine_kernel()(eo, s0f, s1f, w0, w1)

    return out.reshape(B_, S_, D), lb[0, 0]
